# baseline (speedup 1.0000x reference)
import functools

import jax
import jax.numpy as jnp
import numpy as np
from jax.experimental import pallas as pl
from jax.experimental.pallas import tpu as pltpu

S_TILE = 256
V_TILE = 8192


def _stats_kernel(logits_ref, out_ref, m_ref, s0_ref, s1_ref, *, V):
    j = pl.program_id(1)
    nV = pl.num_programs(1)
    x = logits_ref[...]

    col = jax.lax.broadcasted_iota(jnp.int32, x.shape, 1) + j * V_TILE
    x = jnp.where(col < V, x, -1e30)

    @pl.when(j == 0)
    def _():
        m_ref[...] = jnp.full_like(m_ref, -1e30)
        s0_ref[...] = jnp.zeros_like(s0_ref)
        s1_ref[...] = jnp.zeros_like(s1_ref)

    m_old = m_ref[...]
    s0_old = s0_ref[...]
    s1_old = s1_ref[...]

    mc = jnp.max(x, axis=1, keepdims=True)
    m_new = jnp.maximum(m_old, mc)
    scale = jnp.exp(jnp.maximum(m_old - m_new, -100.0))
    t = jnp.maximum(x - m_new, -100.0)
    e = jnp.exp(t)
    s0_chunk = jnp.sum(e, axis=1, keepdims=True)
    s1_chunk = jnp.sum(e * t, axis=1, keepdims=True)

    s0_new = s0_old * scale + s0_chunk
    s1_new = (s1_old + (m_old - m_new) * s0_old) * scale + s1_chunk
    m_ref[...] = m_new
    s0_ref[...] = s0_new
    s1_ref[...] = s1_new

    @pl.when(j == nV - 1)
    def _():
        max_prob = 1.0 / s0_new
        entropy = jnp.log(s0_new) - s1_new / s0_new - (V * 1e-8)
        ent_conf = 1.0 - entropy * np.float32(1.0 / np.log(V))
        out_ref[...] = 0.4 * max_prob + 0.2 * ent_conf


def _combine_kernel(hidden_ref, w1t_ref, b1_ref, w2_ref, b2_ref, mask_ref,
                    part_ref, out_ref, *, S):
    h = hidden_ref[...]

    hh = jnp.dot(h, w1t_ref[...], preferred_element_type=jnp.float32)
    hh = hh + b1_ref[...]
    hh = 0.5 * hh * (1.0 + jax.lax.erf(hh * np.float32(1.0 / np.sqrt(2.0))))
    learned_pre = jnp.sum(hh * w2_ref[...], axis=1, keepdims=True) + b2_ref[...]
    learned = jax.nn.sigmoid(learned_pre)

    ss = jnp.sum(h * h, axis=1, keepdims=True)
    hn = h / jnp.maximum(jnp.sqrt(ss), 1e-12)
    z = jnp.sum(hn[: S - 1, :] * hn[1:, :], axis=1, keepdims=True)
    zero = jnp.zeros((1, 1), dtype=jnp.float32)
    left_full = jnp.concatenate([zero, z], axis=0)
    right_full = jnp.concatenate([z, zero], axis=0)
    idx = jax.lax.broadcasted_iota(jnp.int32, (S, 1), 0)
    count = jnp.where((idx == 0) | (idx == S - 1), 1.0, 2.0)
    context_scores = (left_full + right_full) / count
    context_boost = jax.nn.sigmoid(context_scores * 2.0)

    combined = part_ref[...] + 0.2 * learned + 0.2 * context_boost
    out_ref[...] = combined * mask_ref[...]


def kernel(logits, hidden_states, token_mask, W1, b1, W2, b2):
    B, S, V = logits.shape
    D = hidden_states.shape[-1]
    H = W1.shape[0]
    assert B == 1

    x = logits.reshape(S, V)
    nS = S // S_TILE
    nV = pl.cdiv(V, V_TILE)

    part = pl.pallas_call(
        functools.partial(_stats_kernel, V=V),
        grid=(nS, nV),
        in_specs=[pl.BlockSpec((S_TILE, V_TILE), lambda i, j: (i, j))],
        out_specs=pl.BlockSpec((S_TILE, 1), lambda i, j: (i, 0)),
        out_shape=jax.ShapeDtypeStruct((S, 1), jnp.float32),
        scratch_shapes=[
            pltpu.VMEM((S_TILE, 1), jnp.float32),
            pltpu.VMEM((S_TILE, 1), jnp.float32),
            pltpu.VMEM((S_TILE, 1), jnp.float32),
        ],
        compiler_params=pltpu.CompilerParams(
            dimension_semantics=("parallel", "arbitrary"),
        ),
    )(x)

    h = hidden_states.reshape(S, D)
    w1t = W1.T
    b1r = b1.reshape(1, H)
    w2r = W2.reshape(1, H)
    b2r = b2.reshape(1, 1)
    mask = token_mask.reshape(S, 1).astype(jnp.float32)

    out = pl.pallas_call(
        functools.partial(_combine_kernel, S=S),
        in_specs=[pl.BlockSpec(a.shape, lambda *, _n=a.ndim: (0,) * _n)
                  for a in (h, w1t, b1r, w2r, b2r, mask, part)],
        out_specs=pl.BlockSpec((S, 1), lambda: (0, 0)),
        out_shape=jax.ShapeDtypeStruct((S, 1), jnp.float32),
    )(h, w1t, b1r, w2r, b2r, mask, part)

    return out.reshape(B, S)
